# Initial kernel scaffold; baseline (speedup 1.0000x reference)
import jax
import jax.numpy as jnp
from jax.experimental import pallas as pl


def kernel(corpus, table):
    raise NotImplementedError("write your pallas kernel here")


# speedup vs baseline: 1.8442x; 1.8442x over previous
import jax
import jax.numpy as jnp
from jax import lax
from jax.experimental import pallas as pl
from jax.experimental.pallas import tpu as pltpu
from jax.experimental.pallas import tpu_sc as plsc
import functools

VOCAB = 1000000
DIM = 64
NC = 2
NS = 16
NW = NC * NS

IDX_W = 128
K = 8
GROUP = K * IDX_W


def _make_gather(n_total: int):
    assert n_total % (NW * GROUP) == 0
    b_per_w = n_total // NW
    ngrp = b_per_w // GROUP

    mesh = plsc.VectorSubcoreMesh(core_axis_name="c", subcore_axis_name="s")

    @functools.partial(
        pl.kernel,
        out_type=jax.ShapeDtypeStruct((n_total, DIM), jnp.float32),
        mesh=mesh,
        scratch_types=[
            pltpu.VMEM((K, IDX_W), jnp.int32),
            pltpu.VMEM((GROUP, DIM), jnp.float32),
            pltpu.SemaphoreType.DMA,
        ],
        compiler_params=pltpu.CompilerParams(use_tc_tiling_on_sc=False),
    )
    def gather_kernel(idx_hbm, table_hbm, out_hbm, idx_v, rows_v, sem):
        wid = lax.axis_index("s") * NC + lax.axis_index("c")
        row0 = wid * b_per_w

        @pl.loop(0, ngrp)
        def _group(g):
            pltpu.sync_copy(idx_hbm.at[wid, g], idx_v)
            copies = [
                pltpu.async_copy(
                    table_hbm.at[idx_v.at[j]],
                    rows_v.at[pl.ds(j * IDX_W, IDX_W)],
                    sem,
                )
                for j in range(K)
            ]
            for c in copies:
                c.wait()
            pltpu.sync_copy(rows_v, out_hbm.at[pl.ds(row0 + g * GROUP, GROUP)])

    return gather_kernel


def kernel(corpus, table):
    b, l = corpus.shape
    n_total = b * l
    idx = corpus.reshape(NW, n_total // (NW * GROUP), K, IDX_W).astype(jnp.int32)
    out = _make_gather(n_total)(idx, table)
    return out.reshape(b, l, DIM)


# speedup vs baseline: 1.8726x; 1.0154x over previous
import jax
import jax.numpy as jnp
from jax import lax
from jax.experimental import pallas as pl
from jax.experimental.pallas import tpu as pltpu
from jax.experimental.pallas import tpu_sc as plsc
import functools

VOCAB = 1000000
DIM = 64
NC = 2
NS = 16
NW = NC * NS

IDX_W = 128
K = 5
GROUP = K * IDX_W
NBUF = 2


def _make_gather(n_total: int):
    assert n_total % (NW * GROUP * NBUF) == 0
    b_per_w = n_total // NW
    ngrp = b_per_w // GROUP

    mesh = plsc.VectorSubcoreMesh(core_axis_name="c", subcore_axis_name="s")

    @functools.partial(
        pl.kernel,
        out_type=jax.ShapeDtypeStruct((n_total, DIM), jnp.float32),
        mesh=mesh,
        scratch_types=[
            pltpu.VMEM((NBUF, K, IDX_W), jnp.int32),
            pltpu.VMEM((NBUF, GROUP, DIM), jnp.float32),
            [pltpu.SemaphoreType.DMA] * NBUF,
            [pltpu.SemaphoreType.DMA] * NBUF,
            pltpu.SemaphoreType.DMA,
        ],
        compiler_params=pltpu.CompilerParams(use_tc_tiling_on_sc=False),
    )
    def gather_kernel(idx_hbm, table_hbm, out_hbm, idx_v, rows_v,
                      isems, osems, gsem):
        wid = lax.axis_index("s") * NC + lax.axis_index("c")
        row0 = wid * b_per_w

        for b in range(NBUF):
            pltpu.async_copy(idx_hbm.at[wid, b], idx_v.at[b], isems[b])

        @pl.loop(0, ngrp, step=NBUF)
        def _outer(g0):
            for b in range(NBUF):
                g = g0 + b
                @pl.when(g0 > 0)
                def _():
                    pltpu.make_async_copy(
                        rows_v.at[b],
                        out_hbm.at[pl.ds((g - NBUF) * GROUP + row0, GROUP)],
                        osems[b],
                    ).wait()
                pltpu.make_async_copy(
                    idx_hbm.at[wid, g], idx_v.at[b], isems[b]).wait()
                copies = [
                    pltpu.async_copy(
                        table_hbm.at[idx_v.at[b, j]],
                        rows_v.at[b, pl.ds(j * IDX_W, IDX_W)],
                        gsem,
                    )
                    for j in range(K)
                ]
                for c in copies:
                    c.wait()
                @pl.when(g + NBUF < ngrp)
                def _():
                    pltpu.async_copy(
                        idx_hbm.at[wid, g + NBUF], idx_v.at[b], isems[b])
                pltpu.async_copy(
                    rows_v.at[b],
                    out_hbm.at[pl.ds(g * GROUP + row0, GROUP)],
                    osems[b],
                )

        for b in range(NBUF):
            g = ngrp - NBUF + b
            pltpu.make_async_copy(
                rows_v.at[b],
                out_hbm.at[pl.ds(g * GROUP + row0, GROUP)],
                osems[b],
            ).wait()

    return gather_kernel


def kernel(corpus, table):
    b, l = corpus.shape
    n_total = b * l
    idx = corpus.reshape(NW, n_total // (NW * GROUP), K, IDX_W).astype(jnp.int32)
    out = _make_gather(n_total)(idx, table)
    return out.reshape(b, l, DIM)


# speedup vs baseline: 1.8749x; 1.0012x over previous
import jax
import jax.numpy as jnp
from jax import lax
from jax.experimental import pallas as pl
from jax.experimental.pallas import tpu as pltpu
from jax.experimental.pallas import tpu_sc as plsc
import functools

VOCAB = 1000000
DIM = 64
NC = 2
NS = 16
NW = NC * NS

IDX_W = 640
K = 1
GROUP = K * IDX_W
NBUF = 2


def _make_gather(n_total: int):
    assert n_total % (NW * GROUP * NBUF) == 0
    b_per_w = n_total // NW
    ngrp = b_per_w // GROUP

    mesh = plsc.VectorSubcoreMesh(core_axis_name="c", subcore_axis_name="s")

    @functools.partial(
        pl.kernel,
        out_type=jax.ShapeDtypeStruct((n_total, DIM), jnp.float32),
        mesh=mesh,
        scratch_types=[
            pltpu.VMEM((NBUF, K, IDX_W), jnp.int32),
            pltpu.VMEM((NBUF, GROUP, DIM), jnp.float32),
            [pltpu.SemaphoreType.DMA] * NBUF,
            [pltpu.SemaphoreType.DMA] * NBUF,
            pltpu.SemaphoreType.DMA,
        ],
        compiler_params=pltpu.CompilerParams(use_tc_tiling_on_sc=False),
    )
    def gather_kernel(idx_hbm, table_hbm, out_hbm, idx_v, rows_v,
                      isems, osems, gsem):
        wid = lax.axis_index("s") * NC + lax.axis_index("c")
        row0 = wid * b_per_w

        for b in range(NBUF):
            pltpu.async_copy(idx_hbm.at[wid, b], idx_v.at[b], isems[b])

        @pl.loop(0, ngrp, step=NBUF)
        def _outer(g0):
            for b in range(NBUF):
                g = g0 + b
                @pl.when(g0 > 0)
                def _():
                    pltpu.make_async_copy(
                        rows_v.at[b],
                        out_hbm.at[pl.ds((g - NBUF) * GROUP + row0, GROUP)],
                        osems[b],
                    ).wait()
                pltpu.make_async_copy(
                    idx_hbm.at[wid, g], idx_v.at[b], isems[b]).wait()
                copies = [
                    pltpu.async_copy(
                        table_hbm.at[idx_v.at[b, j]],
                        rows_v.at[b, pl.ds(j * IDX_W, IDX_W)],
                        gsem,
                    )
                    for j in range(K)
                ]
                for c in copies:
                    c.wait()
                @pl.when(g + NBUF < ngrp)
                def _():
                    pltpu.async_copy(
                        idx_hbm.at[wid, g + NBUF], idx_v.at[b], isems[b])
                pltpu.async_copy(
                    rows_v.at[b],
                    out_hbm.at[pl.ds(g * GROUP + row0, GROUP)],
                    osems[b],
                )

        for b in range(NBUF):
            g = ngrp - NBUF + b
            pltpu.make_async_copy(
                rows_v.at[b],
                out_hbm.at[pl.ds(g * GROUP + row0, GROUP)],
                osems[b],
            ).wait()

    return gather_kernel


def kernel(corpus, table):
    b, l = corpus.shape
    n_total = b * l
    idx = corpus.reshape(NW, n_total // (NW * GROUP), K, IDX_W).astype(jnp.int32)
    out = _make_gather(n_total)(idx, table)
    return out.reshape(b, l, DIM)
